# baseline (speedup 1.0000x reference)
import functools

import jax
import jax.numpy as jnp
from jax import lax
from jax.experimental import pallas as pl
from jax.experimental.pallas import tpu as pltpu
from jax.experimental.pallas import tpu_sc as plsc

VOCAB = 1000000
DIM = 64
BATCH = 16384
FIELDS = 26

_INFO = plsc.get_sparse_core_info()
_NC = _INFO.num_cores
_NS = _INFO.num_subcores
_NW = _NC * _NS

_FULLT = VOCAB // 128
_TAIL = VOCAB - _FULLT * 128
_TPW = _FULLT // _NW
_TREM = _FULLT - _TPW * _NW


def _make_format_kernel():
    mesh = plsc.VectorSubcoreMesh(core_axis_name="c", subcore_axis_name="s")

    @functools.partial(
        pl.kernel,
        mesh=mesh,
        out_type=jax.ShapeDtypeStruct((VOCAB * DIM,), jnp.float32),
        scratch_types=[
            pltpu.VMEM((8, 1024), jnp.float32),
            pltpu.VMEM((8, 1024), jnp.float32),
            pltpu.VMEM((8192,), jnp.float32),
            pltpu.VMEM((8192,), jnp.float32),
        ] + [pltpu.SemaphoreType.DMA] * 4,
        compiler_params=pltpu.CompilerParams(
            use_tc_tiling_on_sc=True, needs_layout_passes=False),
    )
    def fmt(wt_hbm, wtail_hbm, out_hbm, in_0, in_1, row_0, row_1, *sems):
        ins, rows = (in_0, in_1), (row_0, row_1)
        isems, osems = sems[:2], sems[2:]
        wid = lax.axis_index("s") * _NC + lax.axis_index("c")
        t0 = wid * _TPW + jnp.minimum(wid, _TREM)
        ncols = _TPW + jnp.where(wid < _TREM, 1, 0)

        lane = lax.iota(jnp.int32, 16)
        l64 = lane * 64

        def fire_in(t, b):
            for g in range(8):
                pltpu.async_copy(
                    wt_hbm.at[pl.ds(8 * g, 8), pl.ds(t * 128, 128)],
                    ins[b].at[:, pl.ds(g * 128, 128)], isems[b],
                )

        def wait_in(b):
            for g in range(8):
                pltpu.make_async_copy(
                    wt_hbm.at[pl.ds(0, 8), pl.ds(0, 128)],
                    ins[b].at[:, pl.ds(0, 128)], isems[b],
                ).wait()

        def wait_out(b):
            pltpu.make_async_copy(
                rows[b], out_hbm.at[pl.ds(0, 8192)], osems[b],
            ).wait()

        def transpose(b):
            for ic in range(8):
                vbase = l64 + ic * 1024
                for g in range(8):
                    for dl in range(8):
                        d = 8 * g + dl
                        vals = ins[b][dl, pl.ds(g * 128 + ic * 16, 16)]
                        plsc.store_scatter(rows[b], [vbase + d], vals)

        fire_in(t0, 0)
        @pl.when(ncols > 1)
        def _():
            fire_in(t0 + 1, 1)

        def body(j, carry):
            for b in range(2):
                cix = 2 * j + b
                c = t0 + cix

                @pl.when(cix < ncols)
                def _():
                    wait_in(b)

                    @pl.when(cix >= 2)
                    def _():
                        wait_out(b)
                    transpose(b)
                    pltpu.async_copy(
                        rows[b], out_hbm.at[pl.ds(c * 8192, 8192)],
                        osems[b],
                    )

                @pl.when(cix + 2 < ncols)
                def _():
                    fire_in(c + 2, b)
            return carry

        lax.fori_loop(0, (_TPW + 2) // 2, body, 0)
        wait_out(0)
        wait_out(1)

        @pl.when(wid == _NW - 1)
        def _():
            for k in range(_TAIL // 8):
                pltpu.async_copy(
                    wtail_hbm.at[pl.ds(8 * k, 8), pl.ds(0, 128)],
                    in_0.at[:, pl.ds(k * 128, 128)], isems[0],
                )
            for k in range(_TAIL // 8):
                pltpu.make_async_copy(
                    wtail_hbm.at[pl.ds(0, 8), pl.ds(0, 128)],
                    in_0.at[:, pl.ds(0, 128)], isems[0],
                ).wait()
            for k in range(_TAIL // 8):
                for dl in range(8):
                    for ic in range(DIM // 16):
                        vals = in_0[dl, pl.ds(k * 128 + ic * 16, 16)]
                        row_0[pl.ds((8 * k + dl) * DIM + ic * 16, 16)] = vals
            pltpu.async_copy(
                row_0.at[pl.ds(0, _TAIL * DIM)],
                out_hbm.at[pl.ds(_FULLT * 128 * DIM, _TAIL * DIM)],
                osems[0],
            )
            pltpu.make_async_copy(
                row_0.at[pl.ds(0, _TAIL * DIM)],
                out_hbm.at[pl.ds(0, _TAIL * DIM)], osems[0],
            ).wait()

    return fmt


_B = BATCH * FIELDS
_BPW = _B // _NW
_CHUNK = 256
_SUB = 128
_NSUB = _CHUNK // _SUB
_NITER = _BPW // _CHUNK
_NB = 4
_NGROUP = _NITER // _NB


def _make_gather_kernel():
    mesh = plsc.VectorSubcoreMesh(core_axis_name="c", subcore_axis_name="s")

    @functools.partial(
        pl.kernel,
        mesh=mesh,
        out_type=jax.ShapeDtypeStruct((_B, DIM), jnp.float32),
        scratch_types=[
            pltpu.VMEM((_BPW,), jnp.int32),
            pltpu.VMEM((_NB, _CHUNK, DIM), jnp.float32),
        ] + [pltpu.SemaphoreType.DMA] * (2 * _NB),
        compiler_params=pltpu.CompilerParams(use_tc_tiling_on_sc=False),
    )
    def emb(x_hbm, table_hbm, out_hbm, idx_v, rows_v, *sems):
        gsems, osems = sems[:_NB], sems[_NB:]
        wid = lax.axis_index("s") * _NC + lax.axis_index("c")
        base = wid * _BPW
        pltpu.sync_copy(x_hbm.at[pl.ds(base, _BPW)], idx_v)

        def fire_gather(c, b):
            for g in range(_NSUB):
                pltpu.async_copy(
                    table_hbm.at[idx_v.at[pl.ds(c * _CHUNK + g * _SUB, _SUB)]],
                    rows_v.at[b].at[pl.ds(g * _SUB, _SUB)],
                    gsems[b],
                )

        def wait_gather(b):
            for g in range(_NSUB):
                pltpu.make_async_copy(
                    table_hbm.at[idx_v.at[pl.ds(g * _SUB, _SUB)]],
                    rows_v.at[b].at[pl.ds(g * _SUB, _SUB)],
                    gsems[b],
                ).wait()

        def wait_out(b):
            pltpu.make_async_copy(
                rows_v.at[b],
                out_hbm.at[pl.ds(base, _CHUNK)],
                osems[b],
            ).wait()

        for b in range(_NB):
            fire_gather(b, b)

        def body(j, carry):
            for b in range(_NB):
                c = j * _NB + b
                wait_gather(b)
                pltpu.async_copy(
                    rows_v.at[b],
                    out_hbm.at[pl.ds(base + c * _CHUNK, _CHUNK)],
                    osems[b],
                )
                bp = (b - 1) % _NB
                cc = c - 1 + _NB
                if b == 0:
                    @pl.when(j >= 1)
                    def _():
                        wait_out(bp)
                        fire_gather(cc, bp)
                else:
                    wait_out(bp)
                    @pl.when(j < _NGROUP - 1)
                    def _():
                        fire_gather(cc, bp)
            return carry

        lax.fori_loop(0, _NGROUP, body, 0)
        wait_out(_NB - 1)

    return emb


_FMT = _make_format_kernel()
_EMB = _make_gather_kernel()


def kernel(x, weight):
    wtail = jnp.pad(weight[_FULLT * 128:, :], ((0, 0), (0, DIM)))
    w_lin = _FMT(weight.T, wtail)
    out = _EMB(x.reshape(-1).astype(jnp.int32), w_lin.reshape(VOCAB, DIM))
    return out.reshape(BATCH, FIELDS, DIM)


# speedup vs baseline: 1.6622x; 1.6622x over previous
import functools

import jax
import jax.numpy as jnp
from jax import lax
from jax.experimental import pallas as pl
from jax.experimental.pallas import tpu as pltpu
from jax.experimental.pallas import tpu_sc as plsc

VOCAB = 1000000
DIM = 64
BATCH = 16384
FIELDS = 26

_INFO = plsc.get_sparse_core_info()
_NC = _INFO.num_cores
_NS = _INFO.num_subcores
_NW = _NC * _NS

_B = BATCH * FIELDS
_BPW = _B // _NW
_CHUNK = 256
_SUB = 128
_NSUB = _CHUNK // _SUB
_NITER = _BPW // _CHUNK
_NB = 4
_NGROUP = _NITER // _NB


def _make_kernel():
    mesh = plsc.VectorSubcoreMesh(core_axis_name="c", subcore_axis_name="s")

    @functools.partial(
        pl.kernel,
        mesh=mesh,
        out_type=jax.ShapeDtypeStruct((_B, DIM), jnp.float32),
        scratch_types=[
            pltpu.VMEM((_BPW,), jnp.int32),
            pltpu.VMEM((_NB, _CHUNK, DIM), jnp.float32),
        ] + [pltpu.SemaphoreType.DMA] * (2 * _NB),
        compiler_params=pltpu.CompilerParams(use_tc_tiling_on_sc=False),
    )
    def emb(x_hbm, table_hbm, out_hbm, idx_v, rows_v, *sems):
        gsems, osems = sems[:_NB], sems[_NB:]
        wid = lax.axis_index("s") * _NC + lax.axis_index("c")
        base = wid * _BPW
        pltpu.sync_copy(x_hbm.at[pl.ds(base, _BPW)], idx_v)

        def fire_gather(c, b):
            for g in range(_NSUB):
                pltpu.async_copy(
                    table_hbm.at[idx_v.at[pl.ds(c * _CHUNK + g * _SUB, _SUB)]],
                    rows_v.at[b].at[pl.ds(g * _SUB, _SUB)],
                    gsems[b],
                )

        def wait_gather(b):
            for g in range(_NSUB):
                pltpu.make_async_copy(
                    table_hbm.at[idx_v.at[pl.ds(g * _SUB, _SUB)]],
                    rows_v.at[b].at[pl.ds(g * _SUB, _SUB)],
                    gsems[b],
                ).wait()

        def wait_out(b):
            pltpu.make_async_copy(
                rows_v.at[b],
                out_hbm.at[pl.ds(base, _CHUNK)],
                osems[b],
            ).wait()

        for b in range(_NB):
            fire_gather(b, b)

        def body(j, carry):
            for b in range(_NB):
                c = j * _NB + b
                wait_gather(b)
                pltpu.async_copy(
                    rows_v.at[b],
                    out_hbm.at[pl.ds(base + c * _CHUNK, _CHUNK)],
                    osems[b],
                )
                bp = (b - 1) % _NB
                cc = c - 1 + _NB
                if b == 0:
                    @pl.when(j >= 1)
                    def _():
                        wait_out(bp)
                        fire_gather(cc, bp)
                else:
                    wait_out(bp)
                    @pl.when(j < _NGROUP - 1)
                    def _():
                        fire_gather(cc, bp)
            return carry

        lax.fori_loop(0, _NGROUP, body, 0)
        wait_out(_NB - 1)

    return emb


_EMB = _make_kernel()


def kernel(x, weight):
    out = _EMB(x.reshape(-1).astype(jnp.int32), weight)
    return out.reshape(BATCH, FIELDS, DIM)
